# baseline (speedup 1.0000x reference)
import functools

import jax
import jax.numpy as jnp
from jax import lax
from jax.experimental import pallas as pl
from jax.experimental.pallas import tpu as pltpu
from jax.experimental.pallas import tpu_sc as plsc

B, D, M = 16384, 128, 100000
NW = 32
ROWS_A = B // NW
CHUNK = 256
UTIL_CHUNK = ((M - B) // NW) // 8 * 8
UTIL_REM = (M - B) - NW * UTIL_CHUNK
UTILIZATION = float(B % M) / float(M)

_mesh = plsc.VectorSubcoreMesh(core_axis_name="c", subcore_axis_name="s")


@functools.partial(
    pl.kernel,
    mesh=_mesh,
    out_type=(
        jax.ShapeDtypeStruct((M, D), jnp.float32),
        jax.ShapeDtypeStruct((M,), jnp.float32),
        jax.ShapeDtypeStruct((16,), jnp.float32),
    ),
    scratch_types=[
        pltpu.VMEM((2, CHUNK, D), jnp.float32),
        pltpu.VMEM((16,), jnp.float32),
        pltpu.VMEM((ROWS_A,), jnp.float32),
        pltpu.VMEM((16,), jnp.float32),
        pltpu.VMEM((UTIL_CHUNK,), jnp.float32),
        pltpu.VMEM((UTIL_REM,), jnp.float32),
        pltpu.SemaphoreType.DMA,
        pltpu.SemaphoreType.DMA,
        pltpu.SemaphoreType.DMA,
        pltpu.SemaphoreType.DMA,
        pltpu.SemaphoreType.DMA,
        pltpu.SemaphoreType.DMA,
        pltpu.SemaphoreType.DMA,
    ],
)
def _sc_scatter(aw_hbm, q_hbm, util_hbm,
                out_mem, out_util, out_scal,
                bufs, q_v, qfill_v, scal_v, util_v, urem_v,
                in_sem0, in_sem1, out_sem0, out_sem1,
                q_sem, uin_sem, uout_sem):
    wid = lax.axis_index("s") * 2 + lax.axis_index("c")
    in_sems = (in_sem0, in_sem1)
    out_sems = (out_sem0, out_sem1)

    h_q = pltpu.async_copy(q_hbm, q_v.at[pl.ds(0, 1)], q_sem)
    u0 = B + wid * UTIL_CHUNK
    h_uin = pltpu.async_copy(util_hbm.at[pl.ds(u0, UTIL_CHUNK)], util_v,
                             uin_sem)

    a0 = wid * ROWS_A
    n = ROWS_A // CHUNK

    def start_in(i):
        b = i % 2
        return pltpu.async_copy(aw_hbm.at[pl.ds(a0 + i * CHUNK, CHUNK)],
                                bufs.at[b], in_sems[b])

    def start_out(i):
        b = i % 2
        return pltpu.async_copy(bufs.at[b],
                                out_mem.at[pl.ds(a0 + i * CHUNK, CHUNK)],
                                out_sems[b])

    pend_out = [None, None]

    def drain_out(b):
        if pend_out[b] is not None:
            pend_out[b].wait()
            pend_out[b] = None

    h_in = [None, None]
    h_in[0] = start_in(0)
    for i in range(n):
        b = i % 2
        if i + 1 < n:
            nb = (i + 1) % 2
            drain_out(nb)
            h_in[nb] = start_in(i + 1)
        h_in[b].wait()
        pend_out[b] = start_out(i)
    drain_out(0)
    drain_out(1)

    h_q.wait()
    q = q_v[...][0]
    qvec = jnp.full((16,), q, dtype=jnp.float32)
    for i in range(ROWS_A // 16):
        qfill_v[pl.ds(i * 16, 16)] = qvec
    h_qout = pltpu.async_copy(qfill_v, out_util.at[pl.ds(a0, ROWS_A)],
                              uout_sem)

    h_uin.wait()
    h_uout = pltpu.async_copy(util_v, out_util.at[pl.ds(u0, UTIL_CHUNK)],
                              uout_sem)

    @pl.when(wid == NW - 1)
    def _copy_util_remainder():
        r0 = B + NW * UTIL_CHUNK
        pltpu.sync_copy(util_hbm.at[pl.ds(r0, UTIL_REM)], urem_v)
        pltpu.sync_copy(urem_v, out_util.at[pl.ds(r0, UTIL_REM)])

    @pl.when(wid == 0)
    def _write_scalars():
        lane = lax.iota(jnp.int32, 16)
        scal_v[...] = jnp.where(lane == 0, jnp.float32(UTILIZATION), qvec)
        pltpu.sync_copy(scal_v, out_scal)

    h_qout.wait()
    h_uout.wait()


TC_CHUNK = 13936
TC_N = (M - B) // TC_CHUNK
TC_NBUF = 6


def _tc_tail_body(head_ref, mem_ref, out_ref, bufs, *sems):
    del head_ref
    in_sems = sems[:TC_NBUF]
    out_sems = sems[TC_NBUF:]

    def start_in(i):
        b = i % TC_NBUF
        return pltpu.async_copy(mem_ref.at[pl.ds(B + i * TC_CHUNK, TC_CHUNK)],
                                bufs.at[b], in_sems[b])

    def start_out(i):
        b = i % TC_NBUF
        return pltpu.async_copy(bufs.at[b],
                                out_ref.at[pl.ds(B + i * TC_CHUNK, TC_CHUNK)],
                                out_sems[b])

    pend_out = [None] * TC_NBUF
    pend_in = [None] * TC_NBUF

    def drain_out(b):
        if pend_out[b] is not None:
            pend_out[b].wait()
            pend_out[b] = None

    for i in range(TC_NBUF - 1):
        pend_in[i % TC_NBUF] = start_in(i)
    for i in range(TC_N):
        b = i % TC_NBUF
        if i + TC_NBUF - 1 < TC_N:
            nb = (i + TC_NBUF - 1) % TC_NBUF
            drain_out(nb)
            pend_in[nb] = start_in(i + TC_NBUF - 1)
        pend_in[b].wait()
        pend_out[b] = start_out(i)
    for b in range(TC_NBUF):
        drain_out(b)


_tc_tail = pl.pallas_call(
    _tc_tail_body,
    in_specs=[pl.BlockSpec(memory_space=pltpu.MemorySpace.HBM),
              pl.BlockSpec(memory_space=pltpu.MemorySpace.HBM)],
    out_specs=pl.BlockSpec(memory_space=pltpu.MemorySpace.HBM),
    out_shape=jax.ShapeDtypeStruct((M, D), jnp.float32),
    input_output_aliases={0: 0},
    scratch_shapes=(
        [pltpu.VMEM((TC_NBUF, TC_CHUNK, D), jnp.float32)]
        + [pltpu.SemaphoreType.DMA] * (2 * TC_NBUF)
    ),
)


def kernel(features, attention_weights, attention_quality,
           memory_attentions, memory_utilities):
    del features
    head_mem, new_util, scal = _sc_scatter(
        attention_weights, attention_quality, memory_utilities)
    new_mem = _tc_tail(head_mem, memory_attentions)
    return (new_mem, new_util, scal[0], scal[1])


# speedup vs baseline: 1.0001x; 1.0001x over previous
import functools

import jax
import jax.numpy as jnp
from jax import lax
from jax.experimental import pallas as pl
from jax.experimental.pallas import tpu as pltpu
from jax.experimental.pallas import tpu_sc as plsc

B, D, M = 16384, 128, 100000
NW = 32
ROWS_A = B // NW
CHUNK = 256
UTIL_CHUNK = ((M - B) // NW) // 8 * 8
UTIL_REM = (M - B) - NW * UTIL_CHUNK
UTILIZATION = float(B % M) / float(M)

_mesh = plsc.VectorSubcoreMesh(core_axis_name="c", subcore_axis_name="s")


@functools.partial(
    pl.kernel,
    mesh=_mesh,
    out_type=(
        jax.ShapeDtypeStruct((M, D), jnp.float32),
        jax.ShapeDtypeStruct((M,), jnp.float32),
        jax.ShapeDtypeStruct((16,), jnp.float32),
    ),
    scratch_types=[
        pltpu.VMEM((2, CHUNK, D), jnp.float32),
        pltpu.VMEM((16,), jnp.float32),
        pltpu.VMEM((ROWS_A,), jnp.float32),
        pltpu.VMEM((16,), jnp.float32),
        pltpu.SemaphoreType.DMA,
        pltpu.SemaphoreType.DMA,
        pltpu.SemaphoreType.DMA,
        pltpu.SemaphoreType.DMA,
        pltpu.SemaphoreType.DMA,
        pltpu.SemaphoreType.DMA,
    ],
)
def _sc_scatter(aw_hbm, q_hbm,
                out_mem, out_util, out_scal,
                bufs, q_v, qfill_v, scal_v,
                in_sem0, in_sem1, out_sem0, out_sem1,
                q_sem, uout_sem):
    wid = lax.axis_index("s") * 2 + lax.axis_index("c")
    in_sems = (in_sem0, in_sem1)
    out_sems = (out_sem0, out_sem1)

    h_q = pltpu.async_copy(q_hbm, q_v.at[pl.ds(0, 1)], q_sem)

    a0 = wid * ROWS_A
    n = ROWS_A // CHUNK

    def start_in(i):
        b = i % 2
        return pltpu.async_copy(aw_hbm.at[pl.ds(a0 + i * CHUNK, CHUNK)],
                                bufs.at[b], in_sems[b])

    def start_out(i):
        b = i % 2
        return pltpu.async_copy(bufs.at[b],
                                out_mem.at[pl.ds(a0 + i * CHUNK, CHUNK)],
                                out_sems[b])

    pend_out = [None, None]

    def drain_out(b):
        if pend_out[b] is not None:
            pend_out[b].wait()
            pend_out[b] = None

    h_in = [None, None]
    h_in[0] = start_in(0)
    for i in range(n):
        b = i % 2
        if i + 1 < n:
            nb = (i + 1) % 2
            drain_out(nb)
            h_in[nb] = start_in(i + 1)
        h_in[b].wait()
        pend_out[b] = start_out(i)
    drain_out(0)
    drain_out(1)

    h_q.wait()
    q = q_v[...][0]
    qvec = jnp.full((16,), q, dtype=jnp.float32)
    for i in range(ROWS_A // 16):
        qfill_v[pl.ds(i * 16, 16)] = qvec
    h_qout = pltpu.async_copy(qfill_v, out_util.at[pl.ds(a0, ROWS_A)],
                              uout_sem)

    @pl.when(wid == 0)
    def _write_scalars():
        lane = lax.iota(jnp.int32, 16)
        scal_v[...] = jnp.where(lane == 0, jnp.float32(UTILIZATION), qvec)
        pltpu.sync_copy(scal_v, out_scal)

    h_qout.wait()


TC_CHUNK = 13936
TC_N = (M - B) // TC_CHUNK
TC_NBUF = 6


def _tc_tail_body(head_ref, mem_ref, uhead_ref, util_ref,
                  out_ref, out_util_ref, bufs, ubuf, *sems):
    del head_ref, uhead_ref
    in_sems = sems[:TC_NBUF]
    out_sems = sems[TC_NBUF:2 * TC_NBUF]
    u_sem = sems[2 * TC_NBUF]

    h_uin = pltpu.async_copy(util_ref.at[pl.ds(B, M - B)], ubuf, u_sem)

    def start_in(i):
        b = i % TC_NBUF
        return pltpu.async_copy(mem_ref.at[pl.ds(B + i * TC_CHUNK, TC_CHUNK)],
                                bufs.at[b], in_sems[b])

    def start_out(i):
        b = i % TC_NBUF
        return pltpu.async_copy(bufs.at[b],
                                out_ref.at[pl.ds(B + i * TC_CHUNK, TC_CHUNK)],
                                out_sems[b])

    pend_out = [None] * TC_NBUF
    pend_in = [None] * TC_NBUF

    def drain_out(b):
        if pend_out[b] is not None:
            pend_out[b].wait()
            pend_out[b] = None

    for i in range(TC_NBUF - 1):
        pend_in[i % TC_NBUF] = start_in(i)
    for i in range(TC_N):
        b = i % TC_NBUF
        if i + TC_NBUF - 1 < TC_N:
            nb = (i + TC_NBUF - 1) % TC_NBUF
            drain_out(nb)
            pend_in[nb] = start_in(i + TC_NBUF - 1)
        pend_in[b].wait()
        pend_out[b] = start_out(i)
    h_uin.wait()
    h_uout = pltpu.async_copy(ubuf, out_util_ref.at[pl.ds(B, M - B)], u_sem)
    for b in range(TC_NBUF):
        drain_out(b)
    h_uout.wait()


_tc_tail = pl.pallas_call(
    _tc_tail_body,
    in_specs=[pl.BlockSpec(memory_space=pltpu.MemorySpace.HBM)] * 4,
    out_specs=(pl.BlockSpec(memory_space=pltpu.MemorySpace.HBM),
               pl.BlockSpec(memory_space=pltpu.MemorySpace.HBM)),
    out_shape=(jax.ShapeDtypeStruct((M, D), jnp.float32),
               jax.ShapeDtypeStruct((M,), jnp.float32)),
    input_output_aliases={0: 0, 2: 1},
    scratch_shapes=(
        [pltpu.VMEM((TC_NBUF, TC_CHUNK, D), jnp.float32),
         pltpu.VMEM((M - B,), jnp.float32)]
        + [pltpu.SemaphoreType.DMA] * (2 * TC_NBUF + 1)
    ),
)


def kernel(features, attention_weights, attention_quality,
           memory_attentions, memory_utilities):
    del features
    head_mem, head_util, scal = _sc_scatter(
        attention_weights, attention_quality)
    new_mem, new_util = _tc_tail(head_mem, memory_attentions,
                                 head_util, memory_utilities)
    return (new_mem, new_util, scal[0], scal[1])


# speedup vs baseline: 1.0292x; 1.0291x over previous
import jax
import jax.numpy as jnp
from jax import lax
from jax.experimental import pallas as pl
from jax.experimental.pallas import tpu as pltpu
from jax.experimental.pallas import tpu_sc as plsc
from jax._src.pallas import mpmd as _mpmd

B, D, M = 16384, 128, 100000
NW = 32
ROWS_A = B // NW
CHUNK = 256
UTILIZATION = float(B % M) / float(M)

_mesh = plsc.VectorSubcoreMesh(core_axis_name="c", subcore_axis_name="s")


TC_CHUNK = 13936
TC_N = (M - B) // TC_CHUNK
TC_NBUF = 6


def _tc_tail_body(mem_ref, out_ref, bufs, *sems):
    in_sems = sems[:TC_NBUF]
    out_sems = sems[TC_NBUF:]

    def start_in(i):
        b = i % TC_NBUF
        return pltpu.async_copy(mem_ref.at[pl.ds(B + i * TC_CHUNK, TC_CHUNK)],
                                bufs.at[b], in_sems[b])

    def start_out(i):
        b = i % TC_NBUF
        return pltpu.async_copy(bufs.at[b],
                                out_ref.at[pl.ds(B + i * TC_CHUNK, TC_CHUNK)],
                                out_sems[b])

    pend_out = [None] * TC_NBUF
    pend_in = [None] * TC_NBUF

    def drain_out(b):
        if pend_out[b] is not None:
            pend_out[b].wait()
            pend_out[b] = None

    for i in range(TC_NBUF - 1):
        pend_in[i % TC_NBUF] = start_in(i)
    for i in range(TC_N):
        b = i % TC_NBUF
        if i + TC_NBUF - 1 < TC_N:
            nb = (i + TC_NBUF - 1) % TC_NBUF
            drain_out(nb)
            pend_in[nb] = start_in(i + TC_NBUF - 1)
        pend_in[b].wait()
        pend_out[b] = start_out(i)
    for b in range(TC_NBUF):
        drain_out(b)


_tc_tail = pl.pallas_call(
    _tc_tail_body,
    in_specs=[pl.BlockSpec(memory_space=pltpu.MemorySpace.HBM)],
    out_specs=pl.BlockSpec(memory_space=pltpu.MemorySpace.HBM),
    out_shape=jax.ShapeDtypeStruct((M, D), jnp.float32),
    scratch_shapes=(
        [pltpu.VMEM((TC_NBUF, TC_CHUNK, D), jnp.float32)]
        + [pltpu.SemaphoreType.DMA] * (2 * TC_NBUF)
    ),
)


def _sc_body(aw_hbm, q_hbm, mem_hbm, util_hbm,
             out_mem, out_util, out_scal,
             bufs, q_v, qfill_v, scal_v,
             in_sem0, in_sem1, out_sem0, out_sem1, q_sem, uout_sem):
    del mem_hbm, util_hbm
    wid = lax.axis_index("s") * 2 + lax.axis_index("c")
    in_sems = (in_sem0, in_sem1)
    out_sems = (out_sem0, out_sem1)

    h_q = pltpu.async_copy(q_hbm, q_v.at[pl.ds(0, 1)], q_sem)

    a0 = wid * ROWS_A
    n = ROWS_A // CHUNK

    def start_in(i):
        b = i % 2
        return pltpu.async_copy(aw_hbm.at[pl.ds(a0 + i * CHUNK, CHUNK)],
                                bufs.at[b], in_sems[b])

    def start_out(i):
        b = i % 2
        return pltpu.async_copy(bufs.at[b],
                                out_mem.at[pl.ds(a0 + i * CHUNK, CHUNK)],
                                out_sems[b])

    pend_out = [None, None]

    def drain_out(b):
        if pend_out[b] is not None:
            pend_out[b].wait()
            pend_out[b] = None

    h_in = [None, None]
    h_in[0] = start_in(0)
    for i in range(n):
        b = i % 2
        if i + 1 < n:
            nb = (i + 1) % 2
            drain_out(nb)
            h_in[nb] = start_in(i + 1)
        h_in[b].wait()
        pend_out[b] = start_out(i)
    drain_out(0)
    drain_out(1)

    h_q.wait()
    q = q_v[...][0]
    qvec = jnp.full((16,), q, dtype=jnp.float32)
    for i in range(ROWS_A // 16):
        qfill_v[pl.ds(i * 16, 16)] = qvec
    h_qout = pltpu.async_copy(qfill_v, out_util.at[pl.ds(a0, ROWS_A)],
                              uout_sem)

    @pl.when(wid == 0)
    def _write_scalars():
        lane = lax.iota(jnp.int32, 16)
        scal_v[...] = jnp.where(lane == 0, jnp.float32(UTILIZATION), qvec)
        pltpu.sync_copy(scal_v, out_scal)

    h_qout.wait()


_sc_update = _mpmd._mpmd_map(
    [(_mesh, _sc_body)],
    out_types=(
        jax.ShapeDtypeStruct((M, D), jnp.float32),
        jax.ShapeDtypeStruct((M,), jnp.float32),
        jax.ShapeDtypeStruct((16,), jnp.float32),
    ),
    input_output_aliases={2: 0, 3: 1},
    scratch_types=[
        pltpu.VMEM((2, CHUNK, D), jnp.float32),
        pltpu.VMEM((16,), jnp.float32),
        pltpu.VMEM((ROWS_A,), jnp.float32),
        pltpu.VMEM((16,), jnp.float32),
        pltpu.SemaphoreType.DMA,
        pltpu.SemaphoreType.DMA,
        pltpu.SemaphoreType.DMA,
        pltpu.SemaphoreType.DMA,
        pltpu.SemaphoreType.DMA,
        pltpu.SemaphoreType.DMA,
    ],
)


def kernel(features, attention_weights, attention_quality,
           memory_attentions, memory_utilities):
    del features
    tail_filled = _tc_tail(memory_attentions)
    new_mem, new_util, scal = _sc_update(
        attention_weights, attention_quality, tail_filled,
        memory_utilities)
    return (new_mem, new_util, scal[0], scal[1])
